# Initial kernel scaffold; baseline (speedup 1.0000x reference)
import jax
import jax.numpy as jnp
from jax.experimental import pallas as pl


def kernel(x, edge_index, W0, a_src0, a_dst0, b0, g0, beta0, W1, a_src1, a_dst1, b1, g1, beta1):
    raise NotImplementedError("write your pallas kernel here")


# speedup vs baseline: 14.0148x; 14.0148x over previous
import functools
import jax
import jax.numpy as jnp
from jax import lax
from jax.experimental import pallas as pl
from jax.experimental.pallas import tpu as pltpu
from jax.experimental.pallas import tpu_sc as plsc

F32 = jnp.float32
I32 = jnp.int32

D = 128
DEXT = 144
NC, NS, L = 2, 16, 16
NW = NC * NS
B = 128


def _round_up(a, m):
    return (a + m - 1) // m * m


def _pre_body(n, npad, x_ref, w_ref, as_ref, ad_ref, hext_ref, adt_ref):
    h = jnp.dot(x_ref[...], w_ref[...].T, preferred_element_type=F32)
    a_s = jnp.sum(h * as_ref[...], axis=1, keepdims=True)
    a_d = jnp.sum(h * ad_ref[...], axis=1, keepdims=True)
    hext_ref[pl.ds(0, n), pl.ds(0, D)] = h
    col16 = lax.broadcasted_iota(I32, (n, 16), 1)
    tail = jnp.where(col16 == 0, 1.0, jnp.where(col16 == 1, a_s, 0.0))
    hext_ref[pl.ds(0, n), pl.ds(D, 16)] = tail.astype(F32)
    hext_ref[pl.ds(n, npad - n), :] = jnp.zeros((npad - n, DEXT), F32)
    adt_ref[pl.ds(0, n), :] = jnp.broadcast_to(a_d, (n, 16))
    adt_ref[pl.ds(n, npad - n), :] = jnp.zeros((npad - n, 16), F32)


def _tc_pre(x, W, a_src, a_dst, npad):
    n = x.shape[0]
    return pl.pallas_call(
        functools.partial(_pre_body, n, npad),
        out_shape=(
            jax.ShapeDtypeStruct((npad, DEXT), F32),
            jax.ShapeDtypeStruct((npad, 16), F32),
        ),
    )(x, W, a_src.reshape(1, D), a_dst.reshape(1, D))


def _sc_body(nb, npad, n, hext, adt, srcs, dsts, out, acc, idxbuf,
             hrows, adbuf, wbuf, sem):
    cid = lax.axis_index("c")
    sid = lax.axis_index("s")
    wid = sid * NC + cid
    rows_per_tile = npad // NS

    pltpu.sync_copy(hext.at[pl.ds(n, B)], hrows)
    for j in range(rows_per_tile // B):
        pltpu.sync_copy(hrows, acc.at[pl.ds(sid * rows_per_tile + j * B, B)])
    plsc.subcore_barrier()

    def batch(j, carry):
        pltpu.sync_copy(srcs.at[wid, j], idxbuf.at[0])
        pltpu.sync_copy(dsts.at[wid, j], idxbuf.at[1])
        pltpu.async_copy(hext.at[idxbuf.at[0]], hrows, sem).wait()
        pltpu.async_copy(adt.at[idxbuf.at[1]], adbuf, sem).wait()
        for g in range(B // L):
            rows = lax.iota(I32, L) + g * L
            asv = plsc.load_gather(hrows, [rows, jnp.full((L,), D + 1, I32)])
            adv = plsc.load_gather(adbuf, [rows, jnp.zeros((L,), I32)])
            s = asv + adv
            wbuf[pl.ds(L, L)] = jnp.exp(jnp.maximum(s, 0.2 * s))
            for r in range(L):
                wr = plsc.load_gather(wbuf, [jnp.full((L,), L + r, I32)])
                e = g * L + r
                for k in range(DEXT // L):
                    hrows[e, pl.ds(k * L, L)] = hrows[e, pl.ds(k * L, L)] * wr
        pltpu.sync_copy(hrows, acc.at[idxbuf.at[1]], add=True)
        return carry

    lax.fori_loop(0, nb, batch, 0)
    plsc.subcore_barrier()

    pltpu.sync_copy(
        acc.at[pl.ds(sid * rows_per_tile, rows_per_tile)],
        out.at[cid, pl.ds(sid * rows_per_tile, rows_per_tile)],
    )


def _sc_edge(hext, adt, srcs, dsts, nb, npad, n):
    mesh = plsc.VectorSubcoreMesh(
        core_axis_name="c", subcore_axis_name="s", num_cores=NC,
        num_subcores=NS)
    return pl.kernel(
        functools.partial(_sc_body, nb, npad, n),
        out_type=jax.ShapeDtypeStruct((NC, npad, DEXT), F32),
        mesh=mesh,
        compiler_params=pltpu.CompilerParams(
            use_tc_tiling_on_sc=False, needs_layout_passes=False),
        scratch_types=[
            pltpu.VMEM_SHARED((npad, DEXT), F32),
            pltpu.VMEM((2, B), I32),
            pltpu.VMEM((B, DEXT), F32),
            pltpu.VMEM((B, 16), F32),
            pltpu.VMEM((2 * L,), F32),
            pltpu.SemaphoreType.DMA,
        ],
    )(hext, adt, srcs, dsts)


def _post_body(n, x_ref, hext_ref, acc_ref, as_ref, ad_ref, b_ref, g_ref,
               beta_ref, out_ref):
    x = x_ref[...]
    h = hext_ref[pl.ds(0, n), pl.ds(0, D)]
    num = (acc_ref[0, pl.ds(0, n), pl.ds(0, D)]
           + acc_ref[1, pl.ds(0, n), pl.ds(0, D)])
    dent = (acc_ref[0, pl.ds(0, n), pl.ds(D, 16)]
            + acc_ref[1, pl.ds(0, n), pl.ds(D, 16)])
    den = dent[:, 0:1]
    a_s = jnp.sum(h * as_ref[...], axis=1, keepdims=True)
    a_d = jnp.sum(h * ad_ref[...], axis=1, keepdims=True)
    s = a_s + a_d
    w_self = jnp.exp(jnp.maximum(s, 0.2 * s))
    x_att = (num + w_self * h) / (den + w_self + 1e-16) + b_ref[...]
    x2 = x + x_att
    mu = jnp.mean(x2, axis=1, keepdims=True)
    var = jnp.mean((x2 - mu) ** 2, axis=1, keepdims=True)
    xn = (x2 - mu) * lax.rsqrt(var + 1e-5) * g_ref[...] + beta_ref[...]
    out_ref[...] = jnp.maximum(xn, 0.0)


def _tc_post(x, hext, acc, a_src, a_dst, b, g, beta):
    n = x.shape[0]
    return pl.pallas_call(
        functools.partial(_post_body, n),
        out_shape=jax.ShapeDtypeStruct((n, D), F32),
    )(x, hext, acc, a_src.reshape(1, D), a_dst.reshape(1, D),
      b.reshape(1, D), g.reshape(1, D), beta.reshape(1, D))


def kernel(x, edge_index, W0, a_src0, a_dst0, b0, g0, beta0,
           W1, a_src1, a_dst1, b1, g1, beta1):
    n = x.shape[0]
    e = edge_index.shape[1]
    npad = _round_up(n + B, NS * B)
    ept = _round_up(_round_up(e, NW) // NW, B)
    nb = ept // B
    epad = NW * nb * B

    src = edge_index[0].astype(I32)
    dst = edge_index[1].astype(I32)
    pad = jnp.full((epad - e,), n, I32)
    srcs = jnp.concatenate([src, pad]).reshape(NW, nb, B)
    dsts = jnp.concatenate([dst, pad]).reshape(NW, nb, B)

    for (W, a_s, a_d, b, g, beta) in (
            (W0, a_src0, a_dst0, b0, g0, beta0),
            (W1, a_src1, a_dst1, b1, g1, beta1)):
        hext, adt = _tc_pre(x, W, a_s, a_d, npad)
        acc = _sc_edge(hext, adt, srcs, dsts, nb, npad, n)
        x = _tc_post(x, hext, acc, a_s, a_d, b, g, beta)
    return x
